# baseline (speedup 1.0000x reference)
import functools

import jax
import jax.numpy as jnp
from jax.experimental import pallas as pl
from jax.experimental.pallas import tpu as pltpu
from jax.experimental.pallas import tpu_sc as plsc

H = 20
OUT = 3

_SC_CORES = 2
_SC_SUBCORES = 16
_NW = _SC_CORES * _SC_SUBCORES
_CHUNK = 128
_NBUF = 4
_YW = 64


def _sc_mesh():
    return plsc.VectorSubcoreMesh(core_axis_name="c", subcore_axis_name="s")


def _precompute_y(x, umat):
    n = x.shape[0]

    def body(x_ref, u_ref, y_ref):
        y_ref[...] = jnp.dot(x_ref[...], u_ref[...],
                             preferred_element_type=jnp.float32)

    return pl.pallas_call(
        body,
        out_shape=jax.ShapeDtypeStruct((n, umat.shape[1]), jnp.float32),
    )(x, umat)


def _gather_rows(table, idx3):
    per_w = idx3.shape[1]
    n_chunks = _NW * per_w
    d = table.shape[1]

    @functools.partial(
        pl.kernel,
        out_type=jax.ShapeDtypeStruct((n_chunks * _CHUNK, d), jnp.float32),
        mesh=_sc_mesh(),
        scratch_types=[pltpu.VMEM((per_w, _CHUNK), jnp.int32)]
        + [pltpu.VMEM((_CHUNK, d), jnp.float32) for _ in range(_NBUF)]
        + [pltpu.SemaphoreType.DMA for _ in range(2 * _NBUF)],
    )
    def k(table_hbm, idx_hbm, out_hbm, idx_v, *bufs_and_sems):
        bufs = bufs_and_sems[:_NBUF]
        gsem = bufs_and_sems[_NBUF:2 * _NBUF]
        wsem = bufs_and_sems[2 * _NBUF:]
        cid = jax.lax.axis_index("c")
        sid = jax.lax.axis_index("s")
        wid = cid * _SC_SUBCORES + sid
        base = wid * per_w

        pltpu.sync_copy(idx_hbm.at[wid], idx_v)

        n_rounds = -(-per_w // _NBUF)

        @pl.loop(0, n_rounds * _NBUF, step=_NBUF)
        def _(j):
            for b in range(_NBUF):
                c = j + b

                @pl.when(c < per_w)
                def _():
                    @pl.when(j > 0)
                    def _():
                        pltpu.make_async_copy(
                            bufs[b], out_hbm.at[pl.ds(0, _CHUNK)],
                            wsem[b]).wait()

                    pltpu.async_copy(
                        table_hbm.at[idx_v.at[c]], bufs[b], gsem[b])

            for b in range(_NBUF):
                c = j + b

                @pl.when(c < per_w)
                def _():
                    pltpu.make_async_copy(
                        table_hbm.at[pl.ds(0, _CHUNK)], bufs[b],
                        gsem[b]).wait()
                    pltpu.async_copy(
                        bufs[b], out_hbm.at[pl.ds((base + c) * _CHUNK,
                                                  _CHUNK)], wsem[b])

        last = (n_rounds - 1) * _NBUF
        for b in range(_NBUF):
            @pl.when(last + b < per_w)
            def _():
                pltpu.make_async_copy(
                    bufs[b], out_hbm.at[pl.ds(0, _CHUNK)], wsem[b]).wait()

    return k(table, idx3)


def _edge_mlp(eat, w1t, b1c, w2t, b2c, w3, b3r):
    e = eat.shape[1]
    be = 2560
    grid = (e // be,)

    def body(ea_ref, w1_ref, b1_ref, w2_ref, b2_ref, w3_ref, b3_ref,
             out_ref):
        at = ea_ref[...]
        ht = jnp.maximum(
            jnp.dot(w1_ref[...], at, preferred_element_type=jnp.float32)
            + b1_ref[...], 0.0)
        ht = jnp.maximum(
            jnp.dot(w2_ref[...], ht, preferred_element_type=jnp.float32)
            + b2_ref[...], 0.0)
        out_ref[...] = jnp.maximum(
            jax.lax.dot_general(ht, w3_ref[...], (((0,), (0,)), ((), ())),
                                preferred_element_type=jnp.float32)
            + b3_ref[...], 0.0)

    full = lambda arr: pl.BlockSpec(arr.shape, lambda i: (0,) * arr.ndim)
    return pl.pallas_call(
        body,
        grid=grid,
        in_specs=[
            pl.BlockSpec((eat.shape[0], be), lambda i: (0, i)),
            full(w1t), full(b1c), full(w2t), full(b2c), full(w3), full(b3r),
        ],
        out_specs=pl.BlockSpec((be, H), lambda i: (i, 0)),
        out_shape=jax.ShapeDtypeStruct((e, H), jnp.float32),
    )(eat, w1t, b1c, w2t, b2c, w3, b3r)


def _combine(h, gz, smatt):
    e = h.shape[0]
    be = 2560
    grid = (e // be,)

    def body(h_ref, gz_ref, s_ref, out_ref):
        hh = h_ref[...]
        g = gz_ref[:, 0:_YW]
        hg = jnp.concatenate(
            [jnp.concatenate([hh, hh, hh], axis=1) * g[:, 0:3 * H],
             g[:, 3 * H:3 * H + OUT],
             jnp.ones((be, 1), jnp.float32)], axis=1)
        out_ref[...] = jax.lax.dot_general(
            s_ref[...], hg, (((1,), (1,)), ((), ())),
            preferred_element_type=jnp.float32)

    return pl.pallas_call(
        body,
        grid=grid,
        in_specs=[
            pl.BlockSpec((be, H), lambda i: (i, 0)),
            pl.BlockSpec((be, gz.shape[1]), lambda i: (i, 0)),
            pl.BlockSpec(smatt.shape, lambda i: (0, 0)),
        ],
        out_specs=pl.BlockSpec((4, be), lambda i: (0, i)),
        out_shape=jax.ShapeDtypeStruct((4, e), jnp.float32),
    )(h, gz, smatt)


def _scatter_messages(msgt4, dst3, zeros_n):
    per_w = dst3.shape[1]
    n = zeros_n.shape[0]

    @functools.partial(
        pl.kernel,
        out_type=jax.ShapeDtypeStruct((_SC_CORES, 4, n), jnp.float32),
        mesh=_sc_mesh(),
        scratch_types=[
            pltpu.VMEM((per_w, _CHUNK), jnp.int32),
            pltpu.VMEM((4, per_w, _CHUNK), jnp.float32),
            pltpu.VMEM_SHARED((n,), jnp.float32),
            pltpu.VMEM_SHARED((n,), jnp.float32),
            pltpu.VMEM_SHARED((n,), jnp.float32),
            pltpu.VMEM_SHARED((n,), jnp.float32),
        ] + [pltpu.SemaphoreType.DMA for _ in range(4)],
    )
    def k(msgt_hbm, dst_hbm, z_hbm, out_hbm, idx_v, val_v,
          acc0, acc1, acc2, acc3, s0, s1, s2, s3):
        cid = jax.lax.axis_index("c")
        sid = jax.lax.axis_index("s")
        wid = cid * _SC_SUBCORES + sid
        accs = [acc0, acc1, acc2, acc3]
        sems = [s0, s1, s2, s3]

        @pl.when(sid == 0)
        def _():
            for o in range(4):
                pltpu.sync_copy(z_hbm, accs[o])

        pltpu.sync_copy(dst_hbm.at[wid], idx_v)
        for o in range(4):
            pltpu.sync_copy(msgt_hbm.at[o, wid], val_v.at[o])
        plsc.subcore_barrier()

        @pl.loop(0, per_w)
        def _(j):
            @pl.when(j > 0)
            def _():
                for o in range(4):
                    pltpu.make_async_copy(
                        val_v.at[o, 0], accs[o].at[idx_v.at[0]],
                        sems[o]).wait()
            for o in range(4):
                pltpu.async_copy(
                    val_v.at[o, j], accs[o].at[idx_v.at[j]], sems[o],
                    add=True)

        for o in range(4):
            pltpu.make_async_copy(
                val_v.at[o, 0], accs[o].at[idx_v.at[0]], sems[o]).wait()

        plsc.subcore_barrier()

        @pl.when(sid == 0)
        def _():
            for o in range(4):
                pltpu.sync_copy(accs[o], out_hbm.at[cid, o])

    return k(msgt4, dst3, zeros_n)


def _finalize(parts, x, rootp, bias3):
    n = x.shape[0]

    def body(p_ref, x_ref, r_ref, b_ref, o_ref):
        s = p_ref[0] + p_ref[1]
        cnt = jnp.maximum(s[:, 3:4], 1.0)
        rt = jnp.dot(x_ref[...], r_ref[...],
                     preferred_element_type=jnp.float32)
        o_ref[...] = s[:, 0:OUT] / cnt + rt[:, 0:OUT] + b_ref[...]

    return pl.pallas_call(
        body,
        out_shape=jax.ShapeDtypeStruct((n, OUT), jnp.float32),
    )(parts, x, rootp, bias3)


def kernel(x, edge_index, edge_attr, W1, b1, W2, b2, W3, b3, W4, b4, root,
           bias):
    n, d = x.shape
    e = edge_attr.shape[0]
    src = edge_index[0].astype(jnp.int32)
    dst = edge_index[1].astype(jnp.int32)

    n_chunks = -(-e // (_CHUNK * _NW)) * _NW
    per_w = n_chunks // _NW
    pad = n_chunks * _CHUNK - e
    spread = jnp.arange(pad, dtype=jnp.int32) % n
    src3 = jnp.concatenate([src, spread]).reshape(_NW, per_w, _CHUNK)
    dst3 = jnp.concatenate([dst, spread]).reshape(_NW, per_w, _CHUNK)

    u2 = W4.reshape(H, d, OUT).transpose(1, 2, 0).reshape(d, H * OUT)
    b4mat = b4.reshape(d, OUT)
    umat = jnp.concatenate(
        [u2, b4mat, jnp.zeros((d, d - 3 * H - OUT), jnp.float32)], axis=1)
    sm = jnp.zeros((_YW, 4), jnp.float32)
    for o in range(OUT):
        sm = sm.at[o * H:(o + 1) * H, o].set(1.0)
        sm = sm.at[3 * H + o, o].set(1.0)
    smatt = sm.at[_YW - 1, 3].set(1.0).T
    rootp = jnp.pad(root, ((0, 0), (0, 1)))
    bias3 = bias.reshape(1, OUT)

    y = _precompute_y(x, umat)
    gz = _gather_rows(y, src3)
    h = _edge_mlp(edge_attr.T, W1.T, b1.reshape(H, 1), W2.T,
                  b2.reshape(H, 1), W3, b3.reshape(1, H))
    msgt = _combine(h, gz, smatt)
    msgt4 = jnp.pad(msgt, ((0, 0), (0, pad))).reshape(
        4, _NW, per_w, _CHUNK)
    parts = _scatter_messages(msgt4, dst3, jnp.zeros((n,), jnp.float32))
    partsn = jnp.transpose(parts, (0, 2, 1))
    return _finalize(partsn, x, rootp, bias3)


# speedup vs baseline: 1.1033x; 1.1033x over previous
import functools

import jax
import jax.numpy as jnp
from jax.experimental import pallas as pl
from jax.experimental.pallas import tpu as pltpu
from jax.experimental.pallas import tpu_sc as plsc

H = 20
OUT = 3

_SC_CORES = 2
_SC_SUBCORES = 16
_NW = _SC_CORES * _SC_SUBCORES
_CHUNK = 128
_NBUF = 4
_YW = 64


def _sc_mesh():
    return plsc.VectorSubcoreMesh(core_axis_name="c", subcore_axis_name="s")


def _precompute_y(x, umat):
    n = x.shape[0]

    def body(x_ref, u_ref, y_ref):
        y_ref[...] = jnp.dot(x_ref[...], u_ref[...],
                             preferred_element_type=jnp.float32)

    return pl.pallas_call(
        body,
        out_shape=jax.ShapeDtypeStruct((n, umat.shape[1]), jnp.float32),
    )(x, umat)


def _gather_rows(table, idx3):
    per_w = idx3.shape[1]
    n_chunks = _NW * per_w
    d = table.shape[1]

    @functools.partial(
        pl.kernel,
        out_type=jax.ShapeDtypeStruct((n_chunks * _CHUNK, d), jnp.float32),
        mesh=_sc_mesh(),
        scratch_types=[pltpu.VMEM((per_w, _CHUNK), jnp.int32)]
        + [pltpu.VMEM((_CHUNK, d), jnp.float32) for _ in range(_NBUF)]
        + [pltpu.SemaphoreType.DMA for _ in range(2 * _NBUF)],
    )
    def k(table_hbm, idx_hbm, out_hbm, idx_v, *bufs_and_sems):
        bufs = bufs_and_sems[:_NBUF]
        gsem = bufs_and_sems[_NBUF:2 * _NBUF]
        wsem = bufs_and_sems[2 * _NBUF:]
        cid = jax.lax.axis_index("c")
        sid = jax.lax.axis_index("s")
        wid = cid * _SC_SUBCORES + sid
        base = wid * per_w

        pltpu.sync_copy(idx_hbm.at[wid], idx_v)

        n_rounds = -(-per_w // _NBUF)

        @pl.loop(0, n_rounds * _NBUF, step=_NBUF)
        def _(j):
            for b in range(_NBUF):
                c = j + b

                @pl.when(c < per_w)
                def _():
                    @pl.when(j > 0)
                    def _():
                        pltpu.make_async_copy(
                            bufs[b], out_hbm.at[pl.ds(0, _CHUNK)],
                            wsem[b]).wait()

                    pltpu.async_copy(
                        table_hbm.at[idx_v.at[c]], bufs[b], gsem[b])

            for b in range(_NBUF):
                c = j + b

                @pl.when(c < per_w)
                def _():
                    pltpu.make_async_copy(
                        table_hbm.at[pl.ds(0, _CHUNK)], bufs[b],
                        gsem[b]).wait()
                    pltpu.async_copy(
                        bufs[b], out_hbm.at[pl.ds((base + c) * _CHUNK,
                                                  _CHUNK)], wsem[b])

        last = (n_rounds - 1) * _NBUF
        for b in range(_NBUF):
            @pl.when(last + b < per_w)
            def _():
                pltpu.make_async_copy(
                    bufs[b], out_hbm.at[pl.ds(0, _CHUNK)], wsem[b]).wait()

    return k(table, idx3)


def _edge_mlp(eat, w1t, b1c, w2t, b2c, w3t, b3c):
    e = eat.shape[1]
    be = 2560
    grid = (e // be,)

    def body(ea_ref, w1_ref, b1_ref, w2_ref, b2_ref, w3_ref, b3_ref,
             out_ref):
        at = ea_ref[...]
        ht = jnp.maximum(
            jnp.dot(w1_ref[...], at, preferred_element_type=jnp.float32)
            + b1_ref[...], 0.0)
        ht = jnp.maximum(
            jnp.dot(w2_ref[...], ht, preferred_element_type=jnp.float32)
            + b2_ref[...], 0.0)
        out_ref[...] = jnp.maximum(
            jnp.dot(w3_ref[...], ht, preferred_element_type=jnp.float32)
            + b3_ref[...], 0.0)

    full = lambda arr: pl.BlockSpec(arr.shape, lambda i: (0,) * arr.ndim)
    return pl.pallas_call(
        body,
        grid=grid,
        in_specs=[
            pl.BlockSpec((eat.shape[0], be), lambda i: (0, i)),
            full(w1t), full(b1c), full(w2t), full(b2c), full(w3t), full(b3c),
        ],
        out_specs=pl.BlockSpec((H, be), lambda i: (0, i)),
        out_shape=jax.ShapeDtypeStruct((H, e), jnp.float32),
    )(eat, w1t, b1c, w2t, b2c, w3t, b3c)


def _combine(ht, gz, smatt, ident):
    e = ht.shape[1]
    be = 2560
    grid = (e // be,)

    def body(h_ref, gz_ref, s_ref, i_ref, out_ref):
        hh = jax.lax.dot_general(
            h_ref[...], i_ref[...], (((0,), (0,)), ((), ())),
            preferred_element_type=jnp.float32)
        g = gz_ref[:, 0:_YW]
        hg = jnp.concatenate(
            [jnp.concatenate([hh, hh, hh], axis=1) * g[:, 0:3 * H],
             g[:, 3 * H:3 * H + OUT],
             jnp.ones((be, 1), jnp.float32)], axis=1)
        out_ref[...] = jax.lax.dot_general(
            s_ref[...], hg, (((1,), (1,)), ((), ())),
            preferred_element_type=jnp.float32)

    return pl.pallas_call(
        body,
        grid=grid,
        in_specs=[
            pl.BlockSpec((H, be), lambda i: (0, i)),
            pl.BlockSpec((be, gz.shape[1]), lambda i: (i, 0)),
            pl.BlockSpec(smatt.shape, lambda i: (0, 0)),
            pl.BlockSpec(ident.shape, lambda i: (0, 0)),
        ],
        out_specs=pl.BlockSpec((4, be), lambda i: (0, i)),
        out_shape=jax.ShapeDtypeStruct((4, e), jnp.float32),
    )(ht, gz, smatt, ident)


def _scatter_messages(msgt4, dst3, zeros_n):
    per_w = dst3.shape[1]
    n = zeros_n.shape[0]

    @functools.partial(
        pl.kernel,
        out_type=jax.ShapeDtypeStruct((_SC_CORES, 4, n), jnp.float32),
        mesh=_sc_mesh(),
        scratch_types=[
            pltpu.VMEM((per_w, _CHUNK), jnp.int32),
            pltpu.VMEM((4, per_w, _CHUNK), jnp.float32),
            pltpu.VMEM_SHARED((n,), jnp.float32),
            pltpu.VMEM_SHARED((n,), jnp.float32),
            pltpu.VMEM_SHARED((n,), jnp.float32),
            pltpu.VMEM_SHARED((n,), jnp.float32),
        ] + [pltpu.SemaphoreType.DMA for _ in range(4)],
    )
    def k(msgt_hbm, dst_hbm, z_hbm, out_hbm, idx_v, val_v,
          acc0, acc1, acc2, acc3, s0, s1, s2, s3):
        cid = jax.lax.axis_index("c")
        sid = jax.lax.axis_index("s")
        wid = cid * _SC_SUBCORES + sid
        accs = [acc0, acc1, acc2, acc3]
        sems = [s0, s1, s2, s3]

        @pl.when(sid == 0)
        def _():
            for o in range(4):
                pltpu.sync_copy(z_hbm, accs[o])

        pltpu.sync_copy(dst_hbm.at[wid], idx_v)
        for o in range(4):
            pltpu.sync_copy(msgt_hbm.at[o, wid], val_v.at[o])
        plsc.subcore_barrier()

        @pl.loop(0, per_w)
        def _(j):
            @pl.when(j > 0)
            def _():
                for o in range(4):
                    pltpu.make_async_copy(
                        val_v.at[o, 0], accs[o].at[idx_v.at[0]],
                        sems[o]).wait()
            for o in range(4):
                pltpu.async_copy(
                    val_v.at[o, j], accs[o].at[idx_v.at[j]], sems[o],
                    add=True)

        for o in range(4):
            pltpu.make_async_copy(
                val_v.at[o, 0], accs[o].at[idx_v.at[0]], sems[o]).wait()

        plsc.subcore_barrier()

        @pl.when(sid == 0)
        def _():
            for o in range(4):
                pltpu.sync_copy(accs[o], out_hbm.at[cid, o])

    return k(msgt4, dst3, zeros_n)


def _finalize(parts, x, rootp, bias3):
    n = x.shape[0]

    def body(p_ref, x_ref, r_ref, b_ref, o_ref):
        s = p_ref[0] + p_ref[1]
        cnt = jnp.maximum(s[:, 3:4], 1.0)
        rt = jnp.dot(x_ref[...], r_ref[...],
                     preferred_element_type=jnp.float32)
        o_ref[...] = s[:, 0:OUT] / cnt + rt[:, 0:OUT] + b_ref[...]

    return pl.pallas_call(
        body,
        out_shape=jax.ShapeDtypeStruct((n, OUT), jnp.float32),
    )(parts, x, rootp, bias3)


def kernel(x, edge_index, edge_attr, W1, b1, W2, b2, W3, b3, W4, b4, root,
           bias):
    n, d = x.shape
    e = edge_attr.shape[0]
    src = edge_index[0].astype(jnp.int32)
    dst = edge_index[1].astype(jnp.int32)

    n_chunks = -(-e // (_CHUNK * _NW)) * _NW
    per_w = n_chunks // _NW
    pad = n_chunks * _CHUNK - e
    spread = jnp.arange(pad, dtype=jnp.int32) % n
    src3 = jnp.concatenate([src, spread]).reshape(_NW, per_w, _CHUNK)
    dst3 = jnp.concatenate([dst, spread]).reshape(_NW, per_w, _CHUNK)

    u2 = W4.reshape(H, d, OUT).transpose(1, 2, 0).reshape(d, H * OUT)
    b4mat = b4.reshape(d, OUT)
    umat = jnp.concatenate(
        [u2, b4mat, jnp.zeros((d, d - 3 * H - OUT), jnp.float32)], axis=1)
    sm = jnp.zeros((_YW, 4), jnp.float32)
    for o in range(OUT):
        sm = sm.at[o * H:(o + 1) * H, o].set(1.0)
        sm = sm.at[3 * H + o, o].set(1.0)
    smatt = sm.at[_YW - 1, 3].set(1.0).T
    rootp = jnp.pad(root, ((0, 0), (0, 1)))
    bias3 = bias.reshape(1, OUT)

    y = _precompute_y(x, umat)
    gz = _gather_rows(y, src3)
    ht = _edge_mlp(edge_attr.T, W1.T, b1.reshape(H, 1), W2.T,
                   b2.reshape(H, 1), W3.T, b3.reshape(H, 1))
    msgt = _combine(ht, gz, smatt, jnp.eye(H, dtype=jnp.float32))
    msgt4 = jnp.pad(msgt, ((0, 0), (0, pad))).reshape(
        4, _NW, per_w, _CHUNK)
    parts = _scatter_messages(msgt4, dst3, jnp.zeros((n,), jnp.float32))
    partsn = jnp.transpose(parts, (0, 2, 1))
    return _finalize(partsn, x, rootp, bias3)


# speedup vs baseline: 1.4607x; 1.3240x over previous
import functools

import jax
import jax.numpy as jnp
from jax.experimental import pallas as pl
from jax.experimental.pallas import tpu as pltpu
from jax.experimental.pallas import tpu_sc as plsc

H = 20
OUT = 3

_SC_CORES = 2
_SC_SUBCORES = 16
_NW = _SC_CORES * _SC_SUBCORES
_CHUNK = 128
_NBUF = 6


def _sc_mesh():
    return plsc.VectorSubcoreMesh(core_axis_name="c", subcore_axis_name="s")


def _gather_rows(table, idx3):
    per_w = idx3.shape[1]
    n_chunks = _NW * per_w
    d = table.shape[1]

    @functools.partial(
        pl.kernel,
        out_type=jax.ShapeDtypeStruct((n_chunks * _CHUNK, d), jnp.float32),
        mesh=_sc_mesh(),
        scratch_types=[pltpu.VMEM((per_w, _CHUNK), jnp.int32)]
        + [pltpu.VMEM((_CHUNK, d), jnp.float32) for _ in range(_NBUF)]
        + [pltpu.SemaphoreType.DMA for _ in range(2 * _NBUF)],
    )
    def k(table_hbm, idx_hbm, out_hbm, idx_v, *bufs_and_sems):
        bufs = bufs_and_sems[:_NBUF]
        gsem = bufs_and_sems[_NBUF:2 * _NBUF]
        wsem = bufs_and_sems[2 * _NBUF:]
        cid = jax.lax.axis_index("c")
        sid = jax.lax.axis_index("s")
        wid = cid * _SC_SUBCORES + sid
        base = wid * per_w

        pltpu.sync_copy(idx_hbm.at[wid], idx_v)

        n_rounds = -(-per_w // _NBUF)

        @pl.loop(0, n_rounds * _NBUF, step=_NBUF)
        def _(j):
            for b in range(_NBUF):
                c = j + b

                @pl.when(c < per_w)
                def _():
                    @pl.when(j > 0)
                    def _():
                        pltpu.make_async_copy(
                            bufs[b], out_hbm.at[pl.ds(0, _CHUNK)],
                            wsem[b]).wait()

                    pltpu.async_copy(
                        table_hbm.at[idx_v.at[c]], bufs[b], gsem[b])

            for b in range(_NBUF):
                c = j + b

                @pl.when(c < per_w)
                def _():
                    pltpu.make_async_copy(
                        table_hbm.at[pl.ds(0, _CHUNK)], bufs[b],
                        gsem[b]).wait()
                    pltpu.async_copy(
                        bufs[b], out_hbm.at[pl.ds((base + c) * _CHUNK,
                                                  _CHUNK)], wsem[b])

        last = (n_rounds - 1) * _NBUF
        for b in range(_NBUF):
            @pl.when(last + b < per_w)
            def _():
                pltpu.make_async_copy(
                    bufs[b], out_hbm.at[pl.ds(0, _CHUNK)], wsem[b]).wait()

    return k(table, idx3)


def _edge_messages(eat, xj, umatt, smatt, w1t, b1c, w2t, b2c, w3t, b3c,
                   col_off, nblk):
    be = 2560
    grid = (nblk,)
    e = nblk * be

    def body(ea_ref, xj_ref, u_ref, s_ref, w1_ref, b1_ref, w2_ref, b2_ref,
             w3_ref, b3_ref, out_ref):
        at = ea_ref[...]
        ht = jnp.maximum(
            jnp.dot(w1_ref[...], at, preferred_element_type=jnp.float32)
            + b1_ref[...], 0.0)
        ht = jnp.maximum(
            jnp.dot(w2_ref[...], ht, preferred_element_type=jnp.float32)
            + b2_ref[...], 0.0)
        ht = jnp.maximum(
            jnp.dot(w3_ref[...], ht, preferred_element_type=jnp.float32)
            + b3_ref[...], 0.0)
        gt = jax.lax.dot_general(
            u_ref[...], xj_ref[...], (((1,), (1,)), ((), ())),
            preferred_element_type=jnp.float32)
        hgt = jnp.concatenate(
            [jnp.concatenate([ht, ht, ht], axis=0) * gt[0:3 * H, :],
             gt[3 * H:3 * H + OUT, :],
             jnp.ones((1, be), jnp.float32)], axis=0)
        out_ref[...] = jnp.dot(s_ref[...], hgt,
                               preferred_element_type=jnp.float32)

    full = lambda arr: pl.BlockSpec(arr.shape, lambda i: (0,) * arr.ndim)
    return pl.pallas_call(
        body,
        grid=grid,
        in_specs=[
            pl.BlockSpec((eat.shape[0], be), lambda i: (0, i + col_off)),
            pl.BlockSpec((be, xj.shape[1]), lambda i: (i, 0)),
            full(umatt), full(smatt),
            full(w1t), full(b1c), full(w2t), full(b2c), full(w3t), full(b3c),
        ],
        out_specs=pl.BlockSpec((4, be), lambda i: (0, i)),
        out_shape=jax.ShapeDtypeStruct((4, e), jnp.float32),
    )(eat, xj, umatt, smatt, w1t, b1c, w2t, b2c, w3t, b3c)


def _scatter_messages(msgt4, dst3, zeros_n):
    per_w = dst3.shape[1]
    n = zeros_n.shape[0]

    @functools.partial(
        pl.kernel,
        out_type=jax.ShapeDtypeStruct((_SC_CORES, 4, n), jnp.float32),
        mesh=_sc_mesh(),
        scratch_types=[
            pltpu.VMEM((per_w, _CHUNK), jnp.int32),
            pltpu.VMEM((4, per_w, _CHUNK), jnp.float32),
            pltpu.VMEM_SHARED((n,), jnp.float32),
            pltpu.VMEM_SHARED((n,), jnp.float32),
            pltpu.VMEM_SHARED((n,), jnp.float32),
            pltpu.VMEM_SHARED((n,), jnp.float32),
        ] + [pltpu.SemaphoreType.DMA for _ in range(4)],
    )
    def k(msgt_hbm, dst_hbm, z_hbm, out_hbm, idx_v, val_v,
          acc0, acc1, acc2, acc3, s0, s1, s2, s3):
        cid = jax.lax.axis_index("c")
        sid = jax.lax.axis_index("s")
        wid = cid * _SC_SUBCORES + sid
        base = wid * per_w
        accs = [acc0, acc1, acc2, acc3]
        sems = [s0, s1, s2, s3]

        @pl.when(sid == 0)
        def _():
            for o in range(4):
                pltpu.sync_copy(z_hbm, accs[o])

        pltpu.sync_copy(dst_hbm.at[wid], idx_v)
        for o in range(4):
            pltpu.sync_copy(msgt_hbm.at[o, wid], val_v.at[o])
        plsc.subcore_barrier()

        @pl.loop(0, per_w)
        def _(j):
            @pl.when(j > 0)
            def _():
                for o in range(4):
                    pltpu.make_async_copy(
                        val_v.at[o, 0], accs[o].at[idx_v.at[0]],
                        sems[o]).wait()
            for o in range(4):
                pltpu.async_copy(
                    val_v.at[o, j], accs[o].at[idx_v.at[j]], sems[o],
                    add=True)

        for o in range(4):
            pltpu.make_async_copy(
                val_v.at[o, 0], accs[o].at[idx_v.at[0]], sems[o]).wait()

        plsc.subcore_barrier()

        @pl.when(sid == 0)
        def _():
            for o in range(4):
                pltpu.sync_copy(accs[o], out_hbm.at[cid, o])

    return k(msgt4, dst3, zeros_n)


def _finalize(parts, x, roott4, biasc):
    n = x.shape[0]

    def body(p_ref, x_ref, r_ref, b_ref, o_ref):
        s = p_ref[0] + p_ref[1]
        cnt = jnp.maximum(s[3:4, :], 1.0)
        rt = jax.lax.dot_general(
            r_ref[...], x_ref[...], (((1,), (1,)), ((), ())),
            preferred_element_type=jnp.float32)
        o_ref[...] = s[0:OUT, :] / cnt + rt[0:OUT, :] + b_ref[0:OUT, :]

    return pl.pallas_call(
        body,
        out_shape=jax.ShapeDtypeStruct((OUT, n), jnp.float32),
    )(parts, x, roott4, biasc)


def kernel(x, edge_index, edge_attr, W1, b1, W2, b2, W3, b3, W4, b4, root,
           bias):
    n, d = x.shape
    e = edge_attr.shape[0]
    src = edge_index[0].astype(jnp.int32)
    dst = edge_index[1].astype(jnp.int32)

    n_chunks = 2 * (-(-e // (2 * _CHUNK * _NW)) * _NW)
    per_w = n_chunks // _NW
    pad = n_chunks * _CHUNK - e
    half = n_chunks * _CHUNK // 2
    spread = jnp.arange(pad, dtype=jnp.int32) % n
    srcp = jnp.concatenate([src, spread])
    src3a = srcp[:half].reshape(_NW, per_w // 2, _CHUNK)
    src3b = srcp[half:].reshape(_NW, per_w // 2, _CHUNK)
    dst3 = jnp.concatenate([dst, spread]).reshape(_NW, per_w, _CHUNK)

    u2 = W4.reshape(H, d, OUT).transpose(1, 2, 0).reshape(d, H * OUT)
    b4mat = b4.reshape(d, OUT)
    umat = jnp.concatenate(
        [u2, b4mat, jnp.zeros((d, 64 - 3 * H - OUT), jnp.float32)], axis=1)
    sm = jnp.zeros((64, 4), jnp.float32)
    for o in range(OUT):
        sm = sm.at[o * H:(o + 1) * H, o].set(1.0)
        sm = sm.at[3 * H + o, o].set(1.0)
    smatt = sm.at[63, 3].set(1.0).T
    roott4 = jnp.pad(root, ((0, 0), (0, 1))).T
    biasc = jnp.pad(bias, (0, 1)).reshape(4, 1)

    be = 2560
    nblk_a = half // be
    nblk_b = (e - half) // be
    weights = (umat.T, smatt, W1.T, b1.reshape(H, 1), W2.T,
               b2.reshape(H, 1), W3.T, b3.reshape(H, 1))

    xj_a = _gather_rows(x, src3a)
    xj_b = _gather_rows(x, src3b)
    msgt_a = _edge_messages(edge_attr.T, xj_a, *weights, 0, nblk_a)
    msgt_b = _edge_messages(edge_attr.T, xj_b, *weights, nblk_a, nblk_b)
    msgt = jnp.concatenate([msgt_a, msgt_b], axis=1)
    msgt4 = jnp.pad(msgt, ((0, 0), (0, pad))).reshape(
        4, _NW, per_w, _CHUNK)
    parts = _scatter_messages(msgt4, dst3, jnp.zeros((n,), jnp.float32))
    return _finalize(parts, x, roott4, biasc).T
